# Initial kernel scaffold; baseline (speedup 1.0000x reference)
import jax
import jax.numpy as jnp
from jax.experimental import pallas as pl


def kernel(x, gate_W, gate_b, W1, b1, W2, b2):
    raise NotImplementedError("write your pallas kernel here")


# speedup vs baseline: 1.3209x; 1.3209x over previous
import functools
import math

import jax
import jax.numpy as jnp
from jax.experimental import pallas as pl
from jax.experimental.pallas import tpu as pltpu

N = 2048
D = 1024
H = 2048
O = 1024
E = 8
N_ACTIVE = 5
TEMP = math.e
LANES = 128

TN = 1024
BH = 512


def _gate_kernel(x_ref, gw_ref, gb_ref, c_ref):
    scores = jnp.dot(x_ref[...], gw_ref[...], preferred_element_type=jnp.float32)
    scores = (scores + gb_ref[...]) * (1.0 / TEMP)
    lane = jax.lax.broadcasted_iota(jnp.int32, scores.shape, 1)
    valid = lane < E
    s = jnp.where(valid, scores, -jnp.inf)
    m = jnp.max(s, axis=1, keepdims=True)
    ex = jnp.where(valid, jnp.exp(s - m), 0.0)
    probs = ex / jnp.sum(ex, axis=1, keepdims=True)
    rank = jnp.zeros(scores.shape, jnp.int32)
    for j in range(E):
        pj = probs[:, j:j + 1]
        rank = rank + jnp.where(pj > probs, 1, 0) \
                    + jnp.where((pj == probs) & (j < lane), 1, 0)
    mask = (rank < N_ACTIVE) & valid
    w = jnp.where(mask, probs, 0.0)
    weights = w / (jnp.sum(w, axis=1, keepdims=True) + 1e-8)
    sign = jnp.where(lane < E // 2, 1.0, -1.0)
    c_ref[...] = weights * sign


def _moe_kernel(x_ref, w1_ref, b1_ref, w2_ref, b2_ref, c_ref, out_ref):
    e = pl.program_id(1)
    hb = pl.program_id(2)

    h = jnp.dot(x_ref[...], w1_ref[0], preferred_element_type=jnp.float32)
    h = jnp.maximum(h + b1_ref[0], 0.0)
    hs = h * c_ref[0]
    y = jnp.dot(hs, w2_ref[0], preferred_element_type=jnp.float32)

    @pl.when((e == 0) & (hb == 0))
    def _():
        out_ref[...] = jnp.zeros_like(out_ref)

    bterm = jnp.where(hb == 0, 1.0, 0.0) * (c_ref[0] * b2_ref[0])
    out_ref[...] += y + bterm


@functools.partial(jax.jit, static_argnames=())
def kernel(x, gate_W, gate_b, W1, b1, W2, b2):
    gwp = jnp.zeros((D, LANES), jnp.float32).at[:, :E].set(gate_W)
    gbp = jnp.zeros((1, LANES), jnp.float32).at[0, :E].set(gate_b)

    c = pl.pallas_call(
        _gate_kernel,
        out_shape=jax.ShapeDtypeStruct((N, LANES), jnp.float32),
    )(x, gwp, gbp)

    cT = jnp.swapaxes(c[:, :E], 0, 1).reshape(E, N, 1)

    nt = N // TN
    nh = H // BH
    out = pl.pallas_call(
        _moe_kernel,
        grid=(nt, E, nh),
        in_specs=[
            pl.BlockSpec((TN, D), lambda t, e, hb: (t, 0)),
            pl.BlockSpec((1, D, BH), lambda t, e, hb: (e, 0, hb)),
            pl.BlockSpec((1, 1, BH), lambda t, e, hb: (e, 0, hb)),
            pl.BlockSpec((1, BH, O), lambda t, e, hb: (e, hb, 0)),
            pl.BlockSpec((1, 1, O), lambda t, e, hb: (e, 0, 0)),
            pl.BlockSpec((1, TN, 1), lambda t, e, hb: (e, t, 0)),
        ],
        out_specs=pl.BlockSpec((TN, O), lambda t, e, hb: (t, 0)),
        out_shape=jax.ShapeDtypeStruct((N, O), jnp.float32),
        compiler_params=pltpu.CompilerParams(
            dimension_semantics=("parallel", "arbitrary", "arbitrary"),
        ),
    )(x, W1, b1.reshape(E, 1, H), W2, b2.reshape(E, 1, O), cT)
    return out
